# Initial kernel scaffold; baseline (speedup 1.0000x reference)
import jax
import jax.numpy as jnp
from jax.experimental import pallas as pl


def kernel(q, k, v):
    raise NotImplementedError("write your pallas kernel here")


# speedup vs baseline: 2.6288x; 2.6288x over previous
import functools
import math

import jax
import jax.numpy as jnp
from jax import lax
from jax.experimental import pallas as pl
from jax.experimental.pallas import tpu as pltpu

BLK = 128


def _corr_kernel(qh_ref, ql_ref, kh_ref, kl_ref, d_ref, p_ref, *,
                 nblk, apn, jpn, ab, jb):
    ap = pl.program_id(1)
    jp = pl.program_id(2)

    @pl.when(jnp.logical_and(ap == 0, jp == 0))
    def _init():
        p_ref[...] = jnp.zeros_like(p_ref)

    qh = qh_ref[0]
    ql = ql_ref[0]
    kh = kh_ref[0]
    kl = kl_ref[0]
    dn = (((1,), (1,)), ((), ()))
    s = lax.dot_general(qh, kh, dn, preferred_element_type=jnp.float32)
    s = s + lax.dot_general(qh, kl, dn, preferred_element_type=jnp.float32)
    s = s + lax.dot_general(ql, kh, dn, preferred_element_type=jnp.float32)

    for ai in range(ab):
        for ji in range(jb):
            c = jnp.mod(jp * jb + ji - ap * ab - ai, nblk)
            blk = s[ai * BLK:(ai + 1) * BLK, ji * BLK:(ji + 1) * BLK]
            p_ref[pl.ds(c * BLK, BLK), :] += blk

    @pl.when(jnp.logical_and(ap == apn - 1, jp == jpn - 1))
    def _finale():
        rows = lax.broadcasted_iota(jnp.int32, (BLK, 2 * BLK), 0)
        for cb in range(nblk):
            a = p_ref[cb * BLK:(cb + 1) * BLK, :]
            nxt = ((cb + 1) % nblk) * BLK
            b2 = p_ref[nxt:nxt + BLK, :]
            r = jnp.concatenate([a, b2], axis=1)
            for bit in range(7):
                sh = 1 << bit
                mask = ((rows >> bit) & 1) == 1
                r = jnp.where(mask, jnp.roll(r, -sh, axis=1), r)
            colsum = jnp.sum(r, axis=0, keepdims=True)
            d_ref[0, :, cb * BLK:(cb + 1) * BLK] = colsum[:, :BLK]


def _corr(qh, ql, kh, kl, interpret=False):
    B, L, C = qh.shape
    nblk = L // BLK
    ab, jb = 8, 4
    apn, jpn = nblk // ab, nblk // jb
    qspec = pl.BlockSpec((1, ab * BLK, C), lambda b, ap, jp: (b, ap, 0))
    kspec = pl.BlockSpec((1, jb * BLK, C), lambda b, ap, jp: (b, jp, 0))
    return pl.pallas_call(
        functools.partial(_corr_kernel, nblk=nblk, apn=apn, jpn=jpn,
                          ab=ab, jb=jb),
        grid=(B, apn, jpn),
        in_specs=[qspec, qspec, kspec, kspec],
        out_specs=pl.BlockSpec((1, 1, L), lambda b, ap, jp: (b, 0, 0)),
        out_shape=jax.ShapeDtypeStruct((B, 1, L), jnp.float32),
        scratch_shapes=[pltpu.VMEM((L, BLK), jnp.float32)],
        interpret=interpret,
    )(qh, ql, kh, kl)


def _agg_kernel(d_ref, v_ref, o_ref, idx_ref, al_ref, *,
                B, L, C, topk):
    b = pl.program_id(0)
    cc = pl.program_id(1)

    @pl.when(jnp.logical_and(b == 0, cc == 0))
    def _select():
        ii = lax.broadcasted_iota(jnp.int32, (1, L), 1)
        m = (d_ref[0:1, :] + d_ref[1:2, :]) * 0.5
        msel = m
        for i in range(topk):
            mx = jnp.max(msel)
            pos = jnp.min(jnp.where(msel == mx, ii, L))
            idx_ref[i] = pos
            msel = jnp.where(ii == pos, -jnp.inf, msel)
        sel = msel == -jnp.inf
        x = d_ref[...] * (1.0 / C)
        xm = jnp.where(sel, x, -jnp.inf)
        mxb = jnp.max(xm, axis=1, keepdims=True)
        e = jnp.where(sel, jnp.exp(x - mxb), 0.0)
        af = e / jnp.sum(e, axis=1, keepdims=True)
        for bb in range(B):
            for i in range(topk):
                pos = idx_ref[i]
                al_ref[bb, i] = jnp.sum(
                    jnp.where(ii == pos, af[bb:bb + 1, :], 0.0))

    vb = v_ref[0]
    o_ref[0] = al_ref[b, 0] * pltpu.roll(vb, L - idx_ref[0], 0)
    for i in range(1, topk):
        o_ref[0] += al_ref[b, i] * pltpu.roll(vb, L - idx_ref[i], 0)


_CW = 128


def _agg(d, v2, topk, interpret=False):
    B, L, C = v2.shape
    vspec = pl.BlockSpec((1, L, _CW), lambda b, cc: (b, 0, cc))
    return pl.pallas_call(
        functools.partial(_agg_kernel, B=B, L=L, C=C, topk=topk),
        grid=(B, C // _CW),
        in_specs=[pl.BlockSpec((B, L), lambda b, cc: (0, 0)), vspec],
        out_specs=vspec,
        out_shape=jax.ShapeDtypeStruct((B, L, C), jnp.float32),
        scratch_shapes=[
            pltpu.SMEM((topk,), jnp.int32),
            pltpu.SMEM((B, topk), jnp.float32),
        ],
        interpret=interpret,
    )(d, v2)


def kernel(q, k, v, interpret=False):
    B, L, H, D = q.shape
    C = H * D
    q2 = q.reshape(B, L, C)
    k2 = k.reshape(B, L, C)
    qh = q2.astype(jnp.bfloat16)
    ql = (q2 - qh.astype(jnp.float32)).astype(jnp.bfloat16)
    kh = k2.astype(jnp.bfloat16)
    kl = (k2 - kh.astype(jnp.float32)).astype(jnp.bfloat16)
    d = _corr(kh, kl, qh, ql, interpret=interpret).reshape(B, L)
    topk = int(math.log(L))
    out = _agg(d, v.reshape(B, L, C), topk, interpret=interpret)
    return out.reshape(B, L, H, D)
